# baseline (speedup 1.0000x reference)
import functools

import jax
import jax.numpy as jnp
from jax import lax
from jax.experimental import pallas as pl
from jax.experimental.pallas import tpu as pltpu
from jax.experimental.pallas import tpu_sc as plsc

NC, NS = 2, 16
NW = NC * NS
EMSIZE = 64
CHUNK = 128


@functools.partial(jax.jit, static_argnums=(2, 3))
def _sc_gather(idx3, table, n_chunks, chunk):
    n_rows = NW * n_chunks * chunk
    per_w = n_chunks * chunk
    mesh = plsc.VectorSubcoreMesh(
        core_axis_name="c", subcore_axis_name="s", num_cores=NC, num_subcores=NS)

    @functools.partial(
        pl.kernel,
        out_type=jax.ShapeDtypeStruct((n_rows, EMSIZE), jnp.float32),
        mesh=mesh,
        scratch_types=[
            pltpu.VMEM((n_chunks, chunk), jnp.int32),
            pltpu.VMEM((chunk, EMSIZE), jnp.float32),
            pltpu.SemaphoreType.DMA,
        ],
        compiler_params=pltpu.CompilerParams(use_tc_tiling_on_sc=False),
    )
    def k(idx_hbm, table_hbm, out_hbm, idx_v, rows_v, sem):
        wid = lax.axis_index("s") * NC + lax.axis_index("c")
        base = wid * per_w
        pltpu.sync_copy(idx_hbm.at[wid], idx_v)

        def body(c, carry):
            pltpu.async_copy(table_hbm.at[idx_v.at[c]], rows_v, sem).wait()
            pltpu.sync_copy(rows_v, out_hbm.at[pl.ds(base + c * chunk, chunk)])
            return carry

        lax.fori_loop(0, n_chunks, body, 0)

    return k(idx3, table)


def kernel(sequence, sequence_char, src_word_table):
    b, l = sequence.shape
    n = b * l
    n_chunks = n // (NW * CHUNK)
    idx3 = sequence.reshape(NW, n_chunks, CHUNK)
    out = _sc_gather(idx3, src_word_table, n_chunks, CHUNK)
    return out.reshape(b, l, EMSIZE)


# speedup vs baseline: 1.0439x; 1.0439x over previous
import functools

import jax
import jax.numpy as jnp
from jax import lax
from jax.experimental import pallas as pl
from jax.experimental.pallas import tpu as pltpu
from jax.experimental.pallas import tpu_sc as plsc

NC, NS = 2, 16
NW = NC * NS
EMSIZE = 64
CHUNK = 128
K = 5


@functools.partial(jax.jit, static_argnums=(2,))
def _sc_gather(idx3, table, n_chunks):
    per_w = n_chunks * CHUNK
    n_rows = NW * per_w
    n_groups = n_chunks // K
    assert n_chunks == n_groups * K and n_groups >= 2 and n_groups % 2 == 0
    mesh = plsc.VectorSubcoreMesh(
        core_axis_name="c", subcore_axis_name="s", num_cores=NC, num_subcores=NS)

    @functools.partial(
        pl.kernel,
        out_type=jax.ShapeDtypeStruct((n_rows, EMSIZE), jnp.float32),
        mesh=mesh,
        scratch_types=[
            pltpu.VMEM((n_chunks, CHUNK), jnp.int32),
            pltpu.VMEM((2, K, CHUNK, EMSIZE), jnp.float32),
            pltpu.SemaphoreType.DMA,
            pltpu.SemaphoreType.DMA,
            pltpu.SemaphoreType.DMA,
            pltpu.SemaphoreType.DMA,
        ],
        compiler_params=pltpu.CompilerParams(use_tc_tiling_on_sc=False),
    )
    def k(idx_hbm, table_hbm, out_hbm, idx_v, rows_v, gsem0, gsem1, wsem0, wsem1):
        wid = lax.axis_index("s") * NC + lax.axis_index("c")
        base = wid * per_w
        pltpu.sync_copy(idx_hbm.at[wid], idx_v)

        def fire(g, s):
            gs = gsem0 if s == 0 else gsem1
            for j in range(K):
                pltpu.async_copy(
                    table_hbm.at[idx_v.at[g * K + j]], rows_v.at[s, j], gs)

        def drain_g(s):
            gs = gsem0 if s == 0 else gsem1
            for j in range(K):
                pltpu.make_async_copy(
                    out_hbm.at[pl.ds(0, CHUNK)], rows_v.at[s, j], gs).wait()

        def fire_wb(g, s):
            ws = wsem0 if s == 0 else wsem1
            for j in range(K):
                c = g * K + j
                pltpu.async_copy(
                    rows_v.at[s, j], out_hbm.at[pl.ds(base + c * CHUNK, CHUNK)], ws)

        def drain_wb(s):
            ws = wsem0 if s == 0 else wsem1
            for j in range(K):
                pltpu.make_async_copy(
                    out_hbm.at[pl.ds(0, CHUNK)], rows_v.at[s, j], ws).wait()

        fire(0, 0)
        fire(1, 1)
        drain_g(0)
        fire_wb(0, 0)

        def body(t, carry):
            for g, s in ((2 * t + 1, 1), (2 * t + 2, 0)):
                drain_wb(1 - s)
                fire(g + 1, 1 - s)
                drain_g(s)
                fire_wb(g, s)
            return carry

        lax.fori_loop(0, (n_groups - 2) // 2, body, 0)

        g_last = n_groups - 1
        drain_wb(0)
        drain_g(1)
        fire_wb(g_last, 1)
        drain_wb(1)

    return k(idx3, table)


def kernel(sequence, sequence_char, src_word_table):
    b, l = sequence.shape
    n = b * l
    n_chunks = n // (NW * CHUNK)
    idx3 = sequence.reshape(NW, n_chunks, CHUNK)
    out = _sc_gather(idx3, src_word_table, n_chunks)
    return out.reshape(b, l, EMSIZE)


# speedup vs baseline: 1.0441x; 1.0002x over previous
import functools

import jax
import jax.numpy as jnp
from jax import lax
from jax.experimental import pallas as pl
from jax.experimental.pallas import tpu as pltpu
from jax.experimental.pallas import tpu_sc as plsc

NC, NS = 2, 16
NW = NC * NS
EMSIZE = 64
CHUNK = 128
K = 5


@functools.partial(jax.jit, static_argnums=(2,))
def _sc_gather(idx, table, n_chunks):
    per_w = n_chunks * CHUNK
    n_rows = NW * per_w
    n_groups = n_chunks // K
    assert n_chunks == n_groups * K and n_groups >= 2 and n_groups % 2 == 0
    mesh = plsc.VectorSubcoreMesh(
        core_axis_name="c", subcore_axis_name="s", num_cores=NC, num_subcores=NS)

    @functools.partial(
        pl.kernel,
        out_type=jax.ShapeDtypeStruct((n_rows, EMSIZE), jnp.float32),
        mesh=mesh,
        scratch_types=[
            pltpu.VMEM((per_w,), jnp.int32),
            pltpu.VMEM((2, K, CHUNK, EMSIZE), jnp.float32),
            pltpu.SemaphoreType.DMA,
            pltpu.SemaphoreType.DMA,
            pltpu.SemaphoreType.DMA,
            pltpu.SemaphoreType.DMA,
        ],
        compiler_params=pltpu.CompilerParams(use_tc_tiling_on_sc=False),
    )
    def k(idx_hbm, table_hbm, out_hbm, idx_v, rows_v, gsem0, gsem1, wsem0, wsem1):
        wid = lax.axis_index("s") * NC + lax.axis_index("c")
        base = wid * per_w
        pltpu.sync_copy(idx_hbm.at[pl.ds(base, per_w)], idx_v)

        def fire(g, s):
            gs = gsem0 if s == 0 else gsem1
            for j in range(K):
                c = g * K + j
                pltpu.async_copy(
                    table_hbm.at[idx_v.at[pl.ds(c * CHUNK, CHUNK)]],
                    rows_v.at[s, j], gs)

        def drain_g(s):
            gs = gsem0 if s == 0 else gsem1
            for j in range(K):
                pltpu.make_async_copy(
                    out_hbm.at[pl.ds(0, CHUNK)], rows_v.at[s, j], gs).wait()

        def fire_wb(g, s):
            ws = wsem0 if s == 0 else wsem1
            for j in range(K):
                c = g * K + j
                pltpu.async_copy(
                    rows_v.at[s, j], out_hbm.at[pl.ds(base + c * CHUNK, CHUNK)], ws)

        def drain_wb(s):
            ws = wsem0 if s == 0 else wsem1
            for j in range(K):
                pltpu.make_async_copy(
                    out_hbm.at[pl.ds(0, CHUNK)], rows_v.at[s, j], ws).wait()

        fire(0, 0)
        fire(1, 1)
        drain_g(0)
        fire_wb(0, 0)

        def body(t, carry):
            for g, s in ((2 * t + 1, 1), (2 * t + 2, 0)):
                drain_wb(1 - s)
                fire(g + 1, 1 - s)
                drain_g(s)
                fire_wb(g, s)
            return carry

        lax.fori_loop(0, (n_groups - 2) // 2, body, 0)

        g_last = n_groups - 1
        drain_wb(0)
        drain_g(1)
        fire_wb(g_last, 1)
        drain_wb(1)

    return k(idx, table)


def kernel(sequence, sequence_char, src_word_table):
    b, l = sequence.shape
    n = b * l
    n_chunks = n // (NW * CHUNK)
    out = _sc_gather(sequence.reshape(n), src_word_table, n_chunks)
    return out.reshape(b, l, EMSIZE)
